# baseline (speedup 1.0000x reference)
import functools
import jax
import jax.numpy as jnp
import numpy as np
from jax import lax
from jax.experimental import pallas as pl
from jax.experimental.pallas import tpu as pltpu
from jax.experimental.pallas import tpu_sc as plsc

N = 10000
E = 160000
D_IN = 256
H1, C1 = 8, 64
H2, C2 = 1, 7

NPAD = 10240
EPAD = 163840
NC, NS, L = 2, 16, 16
ROWS = NPAD // NS
K = 128

_mesh = plsc.VectorSubcoreMesh(core_axis_name="c", subcore_axis_name="s")


def _take(v, idx):
    return lax.gather(
        v, idx[:, None],
        dimension_numbers=lax.GatherDimensionNumbers(
            offset_dims=(), collapsed_slice_dims=(0,), start_index_map=(0,)),
        slice_sizes=(1,),
        mode=lax.GatherScatterMode.PROMISE_IN_BOUNDS)


def _lane_iota():
    return lax.iota(jnp.int32, L)


def _tc_a_body(x_ref, w1_ref, m_ref, ma_ref, p_ref,
               h1_ref, adu_ref, selfnum_ref, selfden_ref):
    h = jnp.dot(x_ref[...], w1_ref[...], preferred_element_type=jnp.float32)
    adu_ref[...] = jnp.dot(h, m_ref[...], preferred_element_type=jnp.float32)
    alpha = jnp.dot(h, ma_ref[...], preferred_element_type=jnp.float32)
    exs = jnp.exp(jnp.where(alpha > 0, alpha, 0.2 * alpha))
    selfden_ref[...] = jnp.dot(exs, p_ref[...],
                               preferred_element_type=jnp.float32)
    for j in range(8):
        h1_ref[j] = h[:, 64 * j:64 * (j + 1)]
        selfnum_ref[j] = h1_ref[j] * exs[:, j:j + 1]


def _tc_a(xp, W1, M, MA, P):
    blk = 256
    grid = NPAD // blk
    return pl.pallas_call(
        _tc_a_body,
        grid=(grid,),
        in_specs=[
            pl.BlockSpec((blk, D_IN), lambda i: (i, 0)),
            pl.BlockSpec((D_IN, 512), lambda i: (0, 0)),
            pl.BlockSpec((512, 16), lambda i: (0, 0)),
            pl.BlockSpec((512, 8), lambda i: (0, 0)),
            pl.BlockSpec((8, 16), lambda i: (0, 0)),
        ],
        out_specs=[
            pl.BlockSpec((8, blk, 64), lambda i: (0, i, 0)),
            pl.BlockSpec((blk, 16), lambda i: (i, 0)),
            pl.BlockSpec((8, blk, 64), lambda i: (0, i, 0)),
            pl.BlockSpec((blk, 16), lambda i: (i, 0)),
        ],
        out_shape=[
            jax.ShapeDtypeStruct((8, NPAD, 64), jnp.float32),
            jax.ShapeDtypeStruct((NPAD, 16), jnp.float32),
            jax.ShapeDtypeStruct((8, NPAD, 64), jnp.float32),
            jax.ShapeDtypeStruct((NPAD, 16), jnp.float32),
        ],
    )(xp, W1, M, MA, P)


def _sc_b_body(srcp, dstp, adu, selfden, zeros16,
               ex1_out, denp_out,
               den_sh, idxs_v, idxd_v, gs_v, gd_v, exb_v, sem):
    c = lax.axis_index("c")
    s = lax.axis_index("s")
    wid = s * NC + c
    row0 = s * ROWS
    @pl.when(c == 0)
    def _():
        pltpu.sync_copy(selfden.at[pl.ds(row0, ROWS)],
                        den_sh.at[pl.ds(row0, ROWS)])
    @pl.when(c != 0)
    def _():
        pltpu.sync_copy(zeros16.at[pl.ds(row0, ROWS)],
                        den_sh.at[pl.ds(row0, ROWS)])
    plsc.subcore_barrier()

    lanes = _lane_iota()
    perm_idx = (lanes + 8) & 15
    nchunks = (EPAD // (NC * NS)) // K

    def chunk(t, _):
        base = wid * (EPAD // (NC * NS)) + t * K
        pltpu.sync_copy(srcp.at[pl.ds(base, K)], idxs_v)
        pltpu.sync_copy(dstp.at[pl.ds(base, K)], idxd_v)
        pltpu.async_copy(adu.at[idxs_v], gs_v, sem).wait()
        pltpu.async_copy(adu.at[idxd_v], gd_v, sem).wait()

        def edge(i, _):
            rs = gs_v[i, :]
            rd = gd_v[i, :]
            al = rs + _take(rd, perm_idx)
            ex = jnp.exp(jnp.maximum(al, 0.2 * al))
            exb_v[i, :] = jnp.where(lanes < 8, ex, 0.0)
            return 0

        lax.fori_loop(0, K, edge, 0)
        pltpu.sync_copy(exb_v, ex1_out.at[pl.ds(base, K)])
        pltpu.sync_copy(exb_v, den_sh.at[idxd_v], add=True)
        return 0

    lax.fori_loop(0, nchunks, chunk, 0)
    plsc.subcore_barrier()
    pltpu.sync_copy(den_sh.at[pl.ds(row0, ROWS)],
                    denp_out.at[pl.ds(c * NPAD + row0, ROWS)])


def _sc_b(srcp, dstp, adu, selfden, zeros16):
    f = pl.kernel(
        _sc_b_body,
        out_type=[
            jax.ShapeDtypeStruct((EPAD, 16), jnp.float32),
            jax.ShapeDtypeStruct((2 * NPAD, 16), jnp.float32),
        ],
        mesh=_mesh,
        scratch_types=[
            pltpu.VMEM_SHARED((NPAD, 16), jnp.float32),
            pltpu.VMEM((K,), jnp.int32),
            pltpu.VMEM((K,), jnp.int32),
            pltpu.VMEM((K, 16), jnp.float32),
            pltpu.VMEM((K, 16), jnp.float32),
            pltpu.VMEM((K, 16), jnp.float32),
            pltpu.SemaphoreType.DMA,
        ],
        compiler_params=pltpu.CompilerParams(use_tc_tiling_on_sc=False),
    )
    return f(srcp, dstp, adu, selfden, zeros16)


KC = 64
_NCHUNK = (EPAD // NS) // KC
_NB = 4


def _sc_c_body(src2d, dst2d, h1flat, ex1, selfnumflat,
               num_out,
               acc_sh, tbl_sh,
               is0, is1, is2, is3, id0, id1, id2, id3,
               hr0, hr1, hr2, hr3, eb0, eb1, eb2, eb3,
               gsem, esem, ssem, isem, idsem):
    c = lax.axis_index("c")
    s = lax.axis_index("s")
    row0 = s * ROWS
    epersub = EPAD // NS
    hr = [hr0, hr1, hr2, hr3]
    eb = [eb0, eb1, eb2, eb3]
    ixs = [is0, is1, is2, is3]
    ixd = [id0, id1, id2, id3]
    T = _NCHUNK

    def idx_load(t, b, sync=False):
        if sync:
            pltpu.sync_copy(src2d.at[s * T + t], ixs[b])
            pltpu.sync_copy(dst2d.at[s * T + t], ixd[b])
        else:
            pltpu.async_copy(src2d.at[s * T + t], ixs[b], isem.at[b])
            pltpu.async_copy(dst2d.at[s * T + t], ixd[b], idsem.at[b])

    def idx_wait(t, b):
        pltpu.make_async_copy(src2d.at[s * T + t], ixs[b], isem.at[b]).wait()
        pltpu.make_async_copy(dst2d.at[s * T + t], ixd[b], idsem.at[b]).wait()

    for p in range(4):
        ch = p * NC + c
        off = ch * NPAD
        pltpu.sync_copy(h1flat.at[pl.ds(off + row0, ROWS)],
                        tbl_sh.at[pl.ds(row0, ROWS)])
        pltpu.sync_copy(selfnumflat.at[pl.ds(off + row0, ROWS)],
                        acc_sh.at[pl.ds(row0, ROWS)])
        plsc.subcore_barrier()
        hh = jnp.full((L,), ch, jnp.int32)

        def issue(t, b):
            pltpu.async_copy(tbl_sh.at[ixs[b]], hr[b], gsem.at[b])
            base = s * epersub + t * KC
            pltpu.async_copy(ex1.at[pl.ds(base, KC)], eb[b], esem.at[b])

        for t0 in range(2):
            idx_load(t0, t0, sync=True)
            issue(t0, t0)
        idx_load(2, 2)
        idx_load(3, 3)

        def group(gg, _):
            for b in range(_NB):
                t = _NB * gg + b
                base = s * epersub + t * KC
                pltpu.make_async_copy(tbl_sh.at[ixs[b]], hr[b],
                                      gsem.at[b]).wait()
                pltpu.make_async_copy(ex1.at[pl.ds(base, KC)], eb[b],
                                      esem.at[b]).wait()

                @plsc.parallel_loop(0, KC, step=1, unroll=2)
                def _edge(i):
                    sc = _take(eb[b][i, :], hh)
                    for j in range(4):
                        hr[b][i, pl.ds(j * L, L)] = \
                            hr[b][i, pl.ds(j * L, L)] * sc

                pltpu.async_copy(hr[b], acc_sh.at[ixd[b]],
                                 ssem.at[b], add=True)

                b2 = (b + 2) % _NB
                @pl.when(t + 2 < T)
                def _():
                    idx_wait(t + 2, b2)
                    issue(t + 2, b2)

                b3 = (b + 3) % _NB
                @pl.when((t >= 1) & (t + 3 < T))
                def _():
                    pltpu.make_async_copy(hr[b3], acc_sh.at[ixd[b3]],
                                          ssem.at[b3]).wait()
                    idx_load(t + 3, b3)
            return 0

        lax.fori_loop(0, T // _NB, group, 0)
        for b in range(_NB):
            pltpu.make_async_copy(hr[b], acc_sh.at[ixd[b]],
                                  ssem.at[b]).wait()
        plsc.subcore_barrier()
        pltpu.sync_copy(acc_sh.at[pl.ds(row0, ROWS)],
                        num_out.at[pl.ds(off + row0, ROWS)])
        plsc.subcore_barrier()


def _sc_c(src2d, dst2d, h1flat, ex1, selfnumflat):
    f = pl.kernel(
        _sc_c_body,
        out_type=jax.ShapeDtypeStruct((8 * NPAD, 64), jnp.float32),
        mesh=_mesh,
        scratch_types=(
            [pltpu.VMEM_SHARED((NPAD, 64), jnp.float32),
             pltpu.VMEM_SHARED((NPAD, 64), jnp.float32)]
            + [pltpu.VMEM((KC,), jnp.int32) for _ in range(8)]
            + [pltpu.VMEM((KC, 64), jnp.float32) for _ in range(4)]
            + [pltpu.VMEM((KC, 16), jnp.float32) for _ in range(4)]
            + [pltpu.SemaphoreType.DMA((_NB,)) for _ in range(5)]
        ),
        compiler_params=pltpu.CompilerParams(use_tc_tiling_on_sc=False),
    )
    return f(src2d, dst2d, h1flat, ex1, selfnumflat)


def _tc_f_body(n0_ref, n1_ref, n2_ref, n3_ref, n4_ref, n5_ref, n6_ref,
               n7_ref, dp0_ref, dp1_ref,
               r16_ref, b1_ref, w2p_ref, a_ref,
               rec2_ref, selfinit2_ref):
    den16 = dp0_ref[...] + dp1_ref[...]
    denb = jnp.dot(den16, r16_ref[...], preferred_element_type=jnp.float32)
    numn = jnp.concatenate(
        [n0_ref[...], n1_ref[...], n2_ref[...], n3_ref[...],
         n4_ref[...], n5_ref[...], n6_ref[...], n7_ref[...]], axis=1)
    o1 = numn / (denb + 1e-16) + b1_ref[...]
    h2pre = jnp.where(o1 > 0, o1, jnp.exp(jnp.minimum(o1, 0.0)) - 1.0)
    h2v = jnp.dot(h2pre, w2p_ref[...], preferred_element_type=jnp.float32)
    sc = jnp.dot(h2v, a_ref[...], preferred_element_type=jnp.float32)
    col = lax.broadcasted_iota(jnp.int32, h2v.shape, 1)
    rec2 = jnp.where(col == 7, 1.0, h2v + sc)
    rec2_ref[...] = rec2
    a2 = rec2[:, 8:9] + rec2[:, 9:10]
    exs2 = jnp.exp(jnp.where(a2 > 0, a2, 0.2 * a2))
    selfinit2_ref[...] = jnp.where(col < 8, rec2 * exs2, 0.0)


def _tc_f(num, denp, R16, b1r, W2p, A):
    blk = 256
    grid = NPAD // blk
    nspec = [pl.BlockSpec((blk, 64), (lambda i, c=c: (c * grid + i, 0)))
             for c in range(8)]
    dspec = [pl.BlockSpec((blk, 16), (lambda i, c=c: (c * grid + i, 0)))
             for c in range(2)]
    return pl.pallas_call(
        _tc_f_body,
        grid=(grid,),
        in_specs=nspec + dspec + [
            pl.BlockSpec((16, 512), lambda i: (0, 0)),
            pl.BlockSpec((1, 512), lambda i: (0, 0)),
            pl.BlockSpec((512, 16), lambda i: (0, 0)),
            pl.BlockSpec((16, 16), lambda i: (0, 0)),
        ],
        out_specs=[
            pl.BlockSpec((blk, 16), lambda i: (i, 0)),
            pl.BlockSpec((blk, 16), lambda i: (i, 0)),
        ],
        out_shape=[
            jax.ShapeDtypeStruct((NPAD, 16), jnp.float32),
            jax.ShapeDtypeStruct((NPAD, 16), jnp.float32),
        ],
    )(num, num, num, num, num, num, num, num, denp, denp, R16, b1r, W2p, A)


def _sc_e_body(srcp, dstp, rec2, selfinit2, zeros16,
               accp_out,
               acc_sh, idxs_v, idxd_v, gs_v, gd_v, msg_v, sem):
    c = lax.axis_index("c")
    s = lax.axis_index("s")
    wid = s * NC + c
    row0 = s * ROWS
    @pl.when(c == 0)
    def _():
        pltpu.sync_copy(selfinit2.at[pl.ds(row0, ROWS)],
                        acc_sh.at[pl.ds(row0, ROWS)])
    @pl.when(c != 0)
    def _():
        pltpu.sync_copy(zeros16.at[pl.ds(row0, ROWS)],
                        acc_sh.at[pl.ds(row0, ROWS)])
    plsc.subcore_barrier()

    lanes = _lane_iota()
    i8 = jnp.full((L,), 8, jnp.int32)
    i9 = jnp.full((L,), 9, jnp.int32)
    nchunks = (EPAD // (NC * NS)) // K

    def chunk(t, _):
        base = wid * (EPAD // (NC * NS)) + t * K
        pltpu.sync_copy(srcp.at[pl.ds(base, K)], idxs_v)
        pltpu.sync_copy(dstp.at[pl.ds(base, K)], idxd_v)
        pltpu.async_copy(rec2.at[idxs_v], gs_v, sem).wait()
        pltpu.async_copy(rec2.at[idxd_v], gd_v, sem).wait()

        def edge(i, _):
            rs = gs_v[i, :]
            rd = gd_v[i, :]
            al = _take(rs, i8) + _take(rd, i9)
            ex = jnp.exp(jnp.maximum(al, 0.2 * al))
            msg_v[i, :] = jnp.where(lanes < 8, rs * ex, 0.0)
            return 0

        lax.fori_loop(0, K, edge, 0)
        pltpu.sync_copy(msg_v, acc_sh.at[idxd_v], add=True)
        return 0

    lax.fori_loop(0, nchunks, chunk, 0)
    plsc.subcore_barrier()
    pltpu.sync_copy(acc_sh.at[pl.ds(row0, ROWS)],
                    accp_out.at[pl.ds(c * NPAD + row0, ROWS)])


def _sc_e(srcp, dstp, rec2, selfinit2, zeros16):
    f = pl.kernel(
        _sc_e_body,
        out_type=jax.ShapeDtypeStruct((2 * NPAD, 16), jnp.float32),
        mesh=_mesh,
        scratch_types=[
            pltpu.VMEM_SHARED((NPAD, 16), jnp.float32),
            pltpu.VMEM((K,), jnp.int32),
            pltpu.VMEM((K,), jnp.int32),
            pltpu.VMEM((K, 16), jnp.float32),
            pltpu.VMEM((K, 16), jnp.float32),
            pltpu.VMEM((K, 16), jnp.float32),
            pltpu.SemaphoreType.DMA,
        ],
        compiler_params=pltpu.CompilerParams(use_tc_tiling_on_sc=False),
    )
    return f(srcp, dstp, rec2, selfinit2, zeros16)


def _tc_g_body(a0_ref, a1_ref, b2_ref, out_ref):
    acc = a0_ref[...] + a1_ref[...]
    o2 = acc / (acc[:, 7:8] + 1e-16) + b2_ref[...]
    col = lax.broadcasted_iota(jnp.int32, o2.shape, 1)
    o2m = jnp.where(col < 7, o2, -jnp.inf)
    m = jnp.max(o2m, axis=1, keepdims=True)
    ex = jnp.where(col < 7, jnp.exp(o2m - m), 0.0)
    ssum = jnp.sum(ex, axis=1, keepdims=True)
    out_ref[...] = o2m - m - jnp.log(ssum)


def _tc_g(accp, b2p):
    blk = 256
    grid = NPAD // blk
    aspec = [pl.BlockSpec((blk, 16), (lambda i, c=c: (c * grid + i, 0)))
             for c in range(2)]
    return pl.pallas_call(
        _tc_g_body,
        grid=(grid,),
        in_specs=aspec + [pl.BlockSpec((1, 16), lambda i: (0, 0))],
        out_specs=pl.BlockSpec((blk, 16), lambda i: (i, 0)),
        out_shape=jax.ShapeDtypeStruct((NPAD, 16), jnp.float32),
    )(accp, accp, b2p)


@jax.jit
def _run(x, src, dst, W1, M, MA, P, R16, b1r, W2p, A, b2p, zeros16):
    xp = jnp.pad(x, ((0, NPAD - N), (0, 0)))
    srcp = jnp.concatenate([src, jnp.full((EPAD - E,), N, jnp.int32)])
    dstp = jnp.concatenate([dst, jnp.full((EPAD - E,), N, jnp.int32)])

    h1t, adu, selfnum, selfden = _tc_a(xp, W1, M, MA, P)
    h1flat = h1t.reshape(8 * NPAD, 64)
    selfnumflat = selfnum.reshape(8 * NPAD, 64)

    ex1, denp = _sc_b(srcp, dstp, adu, selfden, zeros16)
    num = _sc_c(srcp.reshape(EPAD // KC, KC), dstp.reshape(EPAD // KC, KC),
                h1flat, ex1, selfnumflat)
    rec2, selfinit2 = _tc_f(num, denp, R16, b1r, W2p, A)
    accp = _sc_e(srcp, dstp, rec2, selfinit2, zeros16)
    out = _tc_g(accp, b2p)
    return out[:N, :7]


def kernel(x, edge_index, W1, att_src1, att_dst1, b1, W2, att_src2,
           att_dst2, b2):
    as1 = att_src1[0].astype(jnp.float32)
    ad1 = att_dst1[0].astype(jnp.float32)
    a2s = att_src2[0, 0].astype(jnp.float32)
    a2d = att_dst2[0, 0].astype(jnp.float32)

    hsel = np.repeat(np.arange(8), 64)
    eye8 = (hsel[:, None] == np.arange(8)[None, :]).astype(np.float32)
    as_flat = as1.reshape(-1)
    ad_flat = ad1.reshape(-1)
    M = jnp.concatenate([eye8 * as_flat[:, None],
                         eye8 * ad_flat[:, None]], axis=1)
    MA = eye8 * (as_flat + ad_flat)[:, None]
    R16 = jnp.concatenate([jnp.asarray(eye8.T),
                           jnp.zeros((8, 512), jnp.float32)], axis=0)
    P = jnp.concatenate([jnp.eye(8, dtype=jnp.float32),
                         jnp.zeros((8, 8), jnp.float32)], axis=1)
    W2p = jnp.pad(W2.astype(jnp.float32), ((0, 0), (0, 16 - C2)))
    A = jnp.zeros((16, 16), jnp.float32)
    A = A.at[0:7, 8].set(a2s).at[0:7, 9].set(a2d)
    b1r = b1.astype(jnp.float32).reshape(1, 512)
    b2p = jnp.pad(b2.astype(jnp.float32), (0, 9)).reshape(1, 16)
    zeros16 = jnp.zeros((NPAD, 16), jnp.float32)

    src = edge_index[0].astype(jnp.int32)
    dst = edge_index[1].astype(jnp.int32)
    return _run(x, src, dst, W1, M, MA, P, R16,
                b1r, W2p, A, b2p, zeros16)


# speedup vs baseline: 1.3424x; 1.3424x over previous
import functools
import jax
import jax.numpy as jnp
import numpy as np
from jax import lax
from jax.experimental import pallas as pl
from jax.experimental.pallas import tpu as pltpu
from jax.experimental.pallas import tpu_sc as plsc

N = 10000
E = 160000
D_IN = 256
H1, C1 = 8, 64
H2, C2 = 1, 7

NPAD = 10240
EPAD = 163840
NC, NS, L = 2, 16, 16
ROWS = NPAD // NS
K = 128

_mesh = plsc.VectorSubcoreMesh(core_axis_name="c", subcore_axis_name="s")


def _take(v, idx):
    return lax.gather(
        v, idx[:, None],
        dimension_numbers=lax.GatherDimensionNumbers(
            offset_dims=(), collapsed_slice_dims=(0,), start_index_map=(0,)),
        slice_sizes=(1,),
        mode=lax.GatherScatterMode.PROMISE_IN_BOUNDS)


def _lane_iota():
    return lax.iota(jnp.int32, L)


def _tc_a_body(x_ref, w1_ref, m_ref, ma_ref, p_ref,
               h1_ref, adu_ref, selfnum_ref, selfden_ref):
    h = jnp.dot(x_ref[...], w1_ref[...], preferred_element_type=jnp.float32)
    adu_ref[...] = jnp.dot(h, m_ref[...], preferred_element_type=jnp.float32)
    alpha = jnp.dot(h, ma_ref[...], preferred_element_type=jnp.float32)
    exs = jnp.exp(jnp.where(alpha > 0, alpha, 0.2 * alpha))
    selfden_ref[...] = jnp.dot(exs, p_ref[...],
                               preferred_element_type=jnp.float32)
    for j in range(8):
        h1_ref[j] = h[:, 64 * j:64 * (j + 1)]
        selfnum_ref[j] = h1_ref[j] * exs[:, j:j + 1]


def _tc_a(xp, W1, M, MA, P):
    blk = 256
    grid = NPAD // blk
    return pl.pallas_call(
        _tc_a_body,
        grid=(grid,),
        in_specs=[
            pl.BlockSpec((blk, D_IN), lambda i: (i, 0)),
            pl.BlockSpec((D_IN, 512), lambda i: (0, 0)),
            pl.BlockSpec((512, 16), lambda i: (0, 0)),
            pl.BlockSpec((512, 8), lambda i: (0, 0)),
            pl.BlockSpec((8, 16), lambda i: (0, 0)),
        ],
        out_specs=[
            pl.BlockSpec((8, blk, 64), lambda i: (0, i, 0)),
            pl.BlockSpec((blk, 16), lambda i: (i, 0)),
            pl.BlockSpec((8, blk, 64), lambda i: (0, i, 0)),
            pl.BlockSpec((blk, 16), lambda i: (i, 0)),
        ],
        out_shape=[
            jax.ShapeDtypeStruct((8, NPAD, 64), jnp.float32),
            jax.ShapeDtypeStruct((NPAD, 16), jnp.float32),
            jax.ShapeDtypeStruct((8, NPAD, 64), jnp.float32),
            jax.ShapeDtypeStruct((NPAD, 16), jnp.float32),
        ],
    )(xp, W1, M, MA, P)


_KB = 128
_TBE = (EPAD // (NC * NS)) // _KB


def _ring_edge_pass(src2d, dst2d, tbl_hbm, init_hbm, zeros16, acc_sh, tbl_sh,
                    ixs, ixd, gs, gd, gsem, dsem, ssem, isem, idsem,
                    compute, scatter_extra, extra_wait, c, s):
    wid = s * NC + c
    row0 = s * ROWS
    T = _TBE

    @pl.when(c == 0)
    def _():
        pltpu.sync_copy(init_hbm.at[pl.ds(row0, ROWS)],
                        acc_sh.at[pl.ds(row0, ROWS)])
    @pl.when(c != 0)
    def _():
        pltpu.sync_copy(zeros16.at[pl.ds(row0, ROWS)],
                        acc_sh.at[pl.ds(row0, ROWS)])
    pltpu.sync_copy(tbl_hbm.at[pl.ds(row0, ROWS)],
                    tbl_sh.at[pl.ds(row0, ROWS)])
    plsc.subcore_barrier()

    def idx_load(t, b, sync=False):
        if sync:
            pltpu.sync_copy(src2d.at[wid * T + t], ixs[b])
            pltpu.sync_copy(dst2d.at[wid * T + t], ixd[b])
        else:
            pltpu.async_copy(src2d.at[wid * T + t], ixs[b], isem.at[b])
            pltpu.async_copy(dst2d.at[wid * T + t], ixd[b], idsem.at[b])

    def idx_wait(t, b):
        pltpu.make_async_copy(src2d.at[wid * T + t], ixs[b],
                              isem.at[b]).wait()
        pltpu.make_async_copy(dst2d.at[wid * T + t], ixd[b],
                              idsem.at[b]).wait()

    def issue(t, b):
        pltpu.async_copy(tbl_sh.at[ixs[b]], gs[b], gsem.at[b])
        pltpu.async_copy(tbl_sh.at[ixd[b]], gd[b], dsem.at[b])

    for t0 in range(2):
        idx_load(t0, t0, sync=True)
        issue(t0, t0)
    idx_load(2, 2)
    idx_load(3, 3)

    def group(gg, _):
        for b in range(4):
            t = 4 * gg + b
            pltpu.make_async_copy(tbl_sh.at[ixs[b]], gs[b],
                                  gsem.at[b]).wait()
            pltpu.make_async_copy(tbl_sh.at[ixd[b]], gd[b],
                                  dsem.at[b]).wait()
            compute(b)
            pltpu.async_copy(gs[b], acc_sh.at[ixd[b]], ssem.at[b], add=True)
            scatter_extra(t, b)

            b2 = (b + 2) % 4
            @pl.when(t + 2 < T)
            def _():
                idx_wait(t + 2, b2)
                issue(t + 2, b2)

            b3 = (b + 3) % 4
            @pl.when((t >= 1) & (t + 3 < T))
            def _():
                pltpu.make_async_copy(gs[b3], acc_sh.at[ixd[b3]],
                                      ssem.at[b3]).wait()
                extra_wait(t - 1, b3)
                idx_load(t + 3, b3)
        return 0

    lax.fori_loop(0, T // 4, group, 0)
    for b in range(4):
        t = T - 4 + b
        pltpu.make_async_copy(gs[b], acc_sh.at[ixd[b]], ssem.at[b]).wait()
        extra_wait(t, b)
    plsc.subcore_barrier()


def _sc_b_body(src2d, dst2d, adu, selfden, zeros16,
               ex1_out, denp_out,
               den_sh, tbl_sh,
               is0, is1, is2, is3, id0, id1, id2, id3,
               g0, g1, g2, g3, d0, d1, d2, d3,
               gsem, dsem, ssem, xsem, isem, idsem):
    c = lax.axis_index("c")
    s = lax.axis_index("s")
    wid = s * NC + c
    gs = [g0, g1, g2, g3]
    gd = [d0, d1, d2, d3]
    lanes = _lane_iota()
    perm_idx = (lanes + 8) & 15

    def compute(b):
        @plsc.parallel_loop(0, _KB, step=1, unroll=2)
        def _edge(i):
            al = gs[b][i, :] + _take(gd[b][i, :], perm_idx)
            ex = jnp.exp(jnp.maximum(al, 0.2 * al))
            gs[b][i, :] = jnp.where(lanes < 8, ex, 0.0)

    def scatter_extra(t, b):
        base = (wid * _TBE + t) * _KB
        pltpu.async_copy(gs[b], ex1_out.at[pl.ds(base, _KB)], xsem.at[b])

    def extra_wait(t, b):
        base = (wid * _TBE + t) * _KB
        pltpu.make_async_copy(gs[b], ex1_out.at[pl.ds(base, _KB)],
                              xsem.at[b]).wait()

    _ring_edge_pass(src2d, dst2d, adu, selfden, zeros16, den_sh, tbl_sh,
                    [is0, is1, is2, is3], [id0, id1, id2, id3],
                    gs, gd, gsem, dsem, ssem, isem, idsem,
                    compute, scatter_extra, extra_wait, c, s)
    row0 = s * ROWS
    pltpu.sync_copy(den_sh.at[pl.ds(row0, ROWS)],
                    denp_out.at[pl.ds(c * NPAD + row0, ROWS)])


def _sc_b(srcp2d, dstp2d, adu, selfden, zeros16):
    f = pl.kernel(
        _sc_b_body,
        out_type=[
            jax.ShapeDtypeStruct((EPAD, 16), jnp.float32),
            jax.ShapeDtypeStruct((2 * NPAD, 16), jnp.float32),
        ],
        mesh=_mesh,
        scratch_types=(
            [pltpu.VMEM_SHARED((NPAD, 16), jnp.float32),
             pltpu.VMEM_SHARED((NPAD, 16), jnp.float32)]
            + [pltpu.VMEM((_KB,), jnp.int32) for _ in range(8)]
            + [pltpu.VMEM((_KB, 16), jnp.float32) for _ in range(8)]
            + [pltpu.SemaphoreType.DMA((4,)) for _ in range(6)]
        ),
        compiler_params=pltpu.CompilerParams(use_tc_tiling_on_sc=False),
    )
    return f(srcp2d, dstp2d, adu, selfden, zeros16)


KC = 64
_NCHUNK = (EPAD // NS) // KC
_NB = 4


def _sc_c_body(src2d, dst2d, h1flat, ex1, selfnumflat,
               num_out,
               acc_sh, tbl_sh,
               is0, is1, is2, is3, id0, id1, id2, id3,
               hr0, hr1, hr2, hr3, eb0, eb1, eb2, eb3,
               gsem, esem, ssem, isem, idsem):
    c = lax.axis_index("c")
    s = lax.axis_index("s")
    row0 = s * ROWS
    epersub = EPAD // NS
    hr = [hr0, hr1, hr2, hr3]
    eb = [eb0, eb1, eb2, eb3]
    ixs = [is0, is1, is2, is3]
    ixd = [id0, id1, id2, id3]
    T = _NCHUNK

    def idx_load(t, b, sync=False):
        if sync:
            pltpu.sync_copy(src2d.at[s * T + t], ixs[b])
            pltpu.sync_copy(dst2d.at[s * T + t], ixd[b])
        else:
            pltpu.async_copy(src2d.at[s * T + t], ixs[b], isem.at[b])
            pltpu.async_copy(dst2d.at[s * T + t], ixd[b], idsem.at[b])

    def idx_wait(t, b):
        pltpu.make_async_copy(src2d.at[s * T + t], ixs[b], isem.at[b]).wait()
        pltpu.make_async_copy(dst2d.at[s * T + t], ixd[b], idsem.at[b]).wait()

    for p in range(4):
        ch = p * NC + c
        off = ch * NPAD
        pltpu.sync_copy(h1flat.at[pl.ds(off + row0, ROWS)],
                        tbl_sh.at[pl.ds(row0, ROWS)])
        pltpu.sync_copy(selfnumflat.at[pl.ds(off + row0, ROWS)],
                        acc_sh.at[pl.ds(row0, ROWS)])
        plsc.subcore_barrier()
        hh = jnp.full((L,), ch, jnp.int32)

        def issue(t, b):
            pltpu.async_copy(tbl_sh.at[ixs[b]], hr[b], gsem.at[b])
            base = s * epersub + t * KC
            pltpu.async_copy(ex1.at[pl.ds(base, KC)], eb[b], esem.at[b])

        for t0 in range(2):
            idx_load(t0, t0, sync=True)
            issue(t0, t0)
        idx_load(2, 2)
        idx_load(3, 3)

        def group(gg, _):
            for b in range(_NB):
                t = _NB * gg + b
                base = s * epersub + t * KC
                pltpu.make_async_copy(tbl_sh.at[ixs[b]], hr[b],
                                      gsem.at[b]).wait()
                pltpu.make_async_copy(ex1.at[pl.ds(base, KC)], eb[b],
                                      esem.at[b]).wait()

                @plsc.parallel_loop(0, KC, step=1, unroll=2)
                def _edge(i):
                    sc = _take(eb[b][i, :], hh)
                    for j in range(4):
                        hr[b][i, pl.ds(j * L, L)] = \
                            hr[b][i, pl.ds(j * L, L)] * sc

                pltpu.async_copy(hr[b], acc_sh.at[ixd[b]],
                                 ssem.at[b], add=True)

                b2 = (b + 2) % _NB
                @pl.when(t + 2 < T)
                def _():
                    idx_wait(t + 2, b2)
                    issue(t + 2, b2)

                b3 = (b + 3) % _NB
                @pl.when((t >= 1) & (t + 3 < T))
                def _():
                    pltpu.make_async_copy(hr[b3], acc_sh.at[ixd[b3]],
                                          ssem.at[b3]).wait()
                    idx_load(t + 3, b3)
            return 0

        lax.fori_loop(0, T // _NB, group, 0)
        for b in range(_NB):
            pltpu.make_async_copy(hr[b], acc_sh.at[ixd[b]],
                                  ssem.at[b]).wait()
        plsc.subcore_barrier()
        pltpu.sync_copy(acc_sh.at[pl.ds(row0, ROWS)],
                        num_out.at[pl.ds(off + row0, ROWS)])
        plsc.subcore_barrier()


def _sc_c(src2d, dst2d, h1flat, ex1, selfnumflat):
    f = pl.kernel(
        _sc_c_body,
        out_type=jax.ShapeDtypeStruct((8 * NPAD, 64), jnp.float32),
        mesh=_mesh,
        scratch_types=(
            [pltpu.VMEM_SHARED((NPAD, 64), jnp.float32),
             pltpu.VMEM_SHARED((NPAD, 64), jnp.float32)]
            + [pltpu.VMEM((KC,), jnp.int32) for _ in range(8)]
            + [pltpu.VMEM((KC, 64), jnp.float32) for _ in range(4)]
            + [pltpu.VMEM((KC, 16), jnp.float32) for _ in range(4)]
            + [pltpu.SemaphoreType.DMA((_NB,)) for _ in range(5)]
        ),
        compiler_params=pltpu.CompilerParams(use_tc_tiling_on_sc=False),
    )
    return f(src2d, dst2d, h1flat, ex1, selfnumflat)


def _tc_f_body(n0_ref, n1_ref, n2_ref, n3_ref, n4_ref, n5_ref, n6_ref,
               n7_ref, dp0_ref, dp1_ref,
               r16_ref, b1_ref, w2p_ref, a_ref,
               rec2_ref, selfinit2_ref):
    den16 = dp0_ref[...] + dp1_ref[...]
    denb = jnp.dot(den16, r16_ref[...], preferred_element_type=jnp.float32)
    numn = jnp.concatenate(
        [n0_ref[...], n1_ref[...], n2_ref[...], n3_ref[...],
         n4_ref[...], n5_ref[...], n6_ref[...], n7_ref[...]], axis=1)
    o1 = numn / (denb + 1e-16) + b1_ref[...]
    h2pre = jnp.where(o1 > 0, o1, jnp.exp(jnp.minimum(o1, 0.0)) - 1.0)
    h2v = jnp.dot(h2pre, w2p_ref[...], preferred_element_type=jnp.float32)
    sc = jnp.dot(h2v, a_ref[...], preferred_element_type=jnp.float32)
    col = lax.broadcasted_iota(jnp.int32, h2v.shape, 1)
    rec2 = jnp.where(col == 7, 1.0, h2v + sc)
    rec2_ref[...] = rec2
    a2 = rec2[:, 8:9] + rec2[:, 9:10]
    exs2 = jnp.exp(jnp.where(a2 > 0, a2, 0.2 * a2))
    selfinit2_ref[...] = jnp.where(col < 8, rec2 * exs2, 0.0)


def _tc_f(num, denp, R16, b1r, W2p, A):
    blk = 256
    grid = NPAD // blk
    nspec = [pl.BlockSpec((blk, 64), (lambda i, c=c: (c * grid + i, 0)))
             for c in range(8)]
    dspec = [pl.BlockSpec((blk, 16), (lambda i, c=c: (c * grid + i, 0)))
             for c in range(2)]
    return pl.pallas_call(
        _tc_f_body,
        grid=(grid,),
        in_specs=nspec + dspec + [
            pl.BlockSpec((16, 512), lambda i: (0, 0)),
            pl.BlockSpec((1, 512), lambda i: (0, 0)),
            pl.BlockSpec((512, 16), lambda i: (0, 0)),
            pl.BlockSpec((16, 16), lambda i: (0, 0)),
        ],
        out_specs=[
            pl.BlockSpec((blk, 16), lambda i: (i, 0)),
            pl.BlockSpec((blk, 16), lambda i: (i, 0)),
        ],
        out_shape=[
            jax.ShapeDtypeStruct((NPAD, 16), jnp.float32),
            jax.ShapeDtypeStruct((NPAD, 16), jnp.float32),
        ],
    )(num, num, num, num, num, num, num, num, denp, denp, R16, b1r, W2p, A)


def _sc_e_body(src2d, dst2d, rec2, selfinit2, zeros16,
               accp_out,
               acc_sh, tbl_sh,
               is0, is1, is2, is3, id0, id1, id2, id3,
               g0, g1, g2, g3, d0, d1, d2, d3,
               gsem, dsem, ssem, isem, idsem):
    c = lax.axis_index("c")
    s = lax.axis_index("s")
    gs = [g0, g1, g2, g3]
    gd = [d0, d1, d2, d3]
    lanes = _lane_iota()
    i8 = jnp.full((L,), 8, jnp.int32)
    i9 = jnp.full((L,), 9, jnp.int32)

    def compute(b):
        @plsc.parallel_loop(0, _KB, step=1, unroll=2)
        def _edge(i):
            rs = gs[b][i, :]
            al = _take(rs, i8) + _take(gd[b][i, :], i9)
            ex = jnp.exp(jnp.maximum(al, 0.2 * al))
            gs[b][i, :] = jnp.where(lanes < 8, rs * ex, 0.0)

    def scatter_extra(t, b):
        pass

    def extra_wait(t, b):
        pass

    _ring_edge_pass(src2d, dst2d, rec2, selfinit2, zeros16, acc_sh, tbl_sh,
                    [is0, is1, is2, is3], [id0, id1, id2, id3],
                    gs, gd, gsem, dsem, ssem, isem, idsem,
                    compute, scatter_extra, extra_wait, c, s)
    row0 = s * ROWS
    pltpu.sync_copy(acc_sh.at[pl.ds(row0, ROWS)],
                    accp_out.at[pl.ds(c * NPAD + row0, ROWS)])


def _sc_e(srcp2d, dstp2d, rec2, selfinit2, zeros16):
    f = pl.kernel(
        _sc_e_body,
        out_type=jax.ShapeDtypeStruct((2 * NPAD, 16), jnp.float32),
        mesh=_mesh,
        scratch_types=(
            [pltpu.VMEM_SHARED((NPAD, 16), jnp.float32),
             pltpu.VMEM_SHARED((NPAD, 16), jnp.float32)]
            + [pltpu.VMEM((_KB,), jnp.int32) for _ in range(8)]
            + [pltpu.VMEM((_KB, 16), jnp.float32) for _ in range(8)]
            + [pltpu.SemaphoreType.DMA((4,)) for _ in range(5)]
        ),
        compiler_params=pltpu.CompilerParams(use_tc_tiling_on_sc=False),
    )
    return f(srcp2d, dstp2d, rec2, selfinit2, zeros16)


def _tc_g_body(a0_ref, a1_ref, b2_ref, out_ref):
    acc = a0_ref[...] + a1_ref[...]
    o2 = acc / (acc[:, 7:8] + 1e-16) + b2_ref[...]
    col = lax.broadcasted_iota(jnp.int32, o2.shape, 1)
    o2m = jnp.where(col < 7, o2, -jnp.inf)
    m = jnp.max(o2m, axis=1, keepdims=True)
    ex = jnp.where(col < 7, jnp.exp(o2m - m), 0.0)
    ssum = jnp.sum(ex, axis=1, keepdims=True)
    out_ref[...] = o2m - m - jnp.log(ssum)


def _tc_g(accp, b2p):
    blk = 256
    grid = NPAD // blk
    aspec = [pl.BlockSpec((blk, 16), (lambda i, c=c: (c * grid + i, 0)))
             for c in range(2)]
    return pl.pallas_call(
        _tc_g_body,
        grid=(grid,),
        in_specs=aspec + [pl.BlockSpec((1, 16), lambda i: (0, 0))],
        out_specs=pl.BlockSpec((blk, 16), lambda i: (i, 0)),
        out_shape=jax.ShapeDtypeStruct((NPAD, 16), jnp.float32),
    )(accp, accp, b2p)


@jax.jit
def _run(x, src, dst, W1, M, MA, P, R16, b1r, W2p, A, b2p, zeros16):
    xp = jnp.pad(x, ((0, NPAD - N), (0, 0)))
    srcp = jnp.concatenate([src, jnp.full((EPAD - E,), N, jnp.int32)])
    dstp = jnp.concatenate([dst, jnp.full((EPAD - E,), N, jnp.int32)])

    h1t, adu, selfnum, selfden = _tc_a(xp, W1, M, MA, P)
    h1flat = h1t.reshape(8 * NPAD, 64)
    selfnumflat = selfnum.reshape(8 * NPAD, 64)

    sb = srcp.reshape(EPAD // _KB, _KB)
    db = dstp.reshape(EPAD // _KB, _KB)
    ex1, denp = _sc_b(sb, db, adu, selfden, zeros16)
    num = _sc_c(srcp.reshape(EPAD // KC, KC), dstp.reshape(EPAD // KC, KC),
                h1flat, ex1, selfnumflat)
    rec2, selfinit2 = _tc_f(num, denp, R16, b1r, W2p, A)
    accp = _sc_e(sb, db, rec2, selfinit2, zeros16)
    out = _tc_g(accp, b2p)
    return out[:N, :7]


def kernel(x, edge_index, W1, att_src1, att_dst1, b1, W2, att_src2,
           att_dst2, b2):
    as1 = att_src1[0].astype(jnp.float32)
    ad1 = att_dst1[0].astype(jnp.float32)
    a2s = att_src2[0, 0].astype(jnp.float32)
    a2d = att_dst2[0, 0].astype(jnp.float32)

    hsel = np.repeat(np.arange(8), 64)
    eye8 = (hsel[:, None] == np.arange(8)[None, :]).astype(np.float32)
    as_flat = as1.reshape(-1)
    ad_flat = ad1.reshape(-1)
    M = jnp.concatenate([eye8 * as_flat[:, None],
                         eye8 * ad_flat[:, None]], axis=1)
    MA = eye8 * (as_flat + ad_flat)[:, None]
    R16 = jnp.concatenate([jnp.asarray(eye8.T),
                           jnp.zeros((8, 512), jnp.float32)], axis=0)
    P = jnp.concatenate([jnp.eye(8, dtype=jnp.float32),
                         jnp.zeros((8, 8), jnp.float32)], axis=1)
    W2p = jnp.pad(W2.astype(jnp.float32), ((0, 0), (0, 16 - C2)))
    A = jnp.zeros((16, 16), jnp.float32)
    A = A.at[0:7, 8].set(a2s).at[0:7, 9].set(a2d)
    b1r = b1.astype(jnp.float32).reshape(1, 512)
    b2p = jnp.pad(b2.astype(jnp.float32), (0, 9)).reshape(1, 16)
    zeros16 = jnp.zeros((NPAD, 16), jnp.float32)

    src = edge_index[0].astype(jnp.int32)
    dst = edge_index[1].astype(jnp.int32)
    return _run(x, src, dst, W1, M, MA, P, R16,
                b1r, W2p, A, b2p, zeros16)


# speedup vs baseline: 1.4078x; 1.0487x over previous
import functools
import jax
import jax.numpy as jnp
import numpy as np
from jax import lax
from jax.experimental import pallas as pl
from jax.experimental.pallas import tpu as pltpu
from jax.experimental.pallas import tpu_sc as plsc

N = 10000
E = 160000
D_IN = 256
H1, C1 = 8, 64
H2, C2 = 1, 7

NPAD = 10240
EPAD = 163840
NC, NS, L = 2, 16, 16
ROWS = NPAD // NS
K = 128

_mesh = plsc.VectorSubcoreMesh(core_axis_name="c", subcore_axis_name="s")


def _take(v, idx):
    return lax.gather(
        v, idx[:, None],
        dimension_numbers=lax.GatherDimensionNumbers(
            offset_dims=(), collapsed_slice_dims=(0,), start_index_map=(0,)),
        slice_sizes=(1,),
        mode=lax.GatherScatterMode.PROMISE_IN_BOUNDS)


def _lane_iota():
    return lax.iota(jnp.int32, L)


def _tc_a_body(x_ref, w1_ref, m_ref, ma_ref, p_ref,
               h1_ref, adu_ref, selfnum_ref, selfden_ref):
    h = jnp.dot(x_ref[...], w1_ref[...], preferred_element_type=jnp.float32)
    adu_ref[...] = jnp.dot(h, m_ref[...], preferred_element_type=jnp.float32)
    alpha = jnp.dot(h, ma_ref[...], preferred_element_type=jnp.float32)
    exs = jnp.exp(jnp.where(alpha > 0, alpha, 0.2 * alpha))
    selfden_ref[...] = jnp.dot(exs, p_ref[...],
                               preferred_element_type=jnp.float32)
    for j in range(8):
        h1_ref[j] = h[:, 64 * j:64 * (j + 1)]
        selfnum_ref[j] = h1_ref[j] * exs[:, j:j + 1]


def _tc_a(xp, W1, M, MA, P):
    blk = 256
    grid = NPAD // blk
    return pl.pallas_call(
        _tc_a_body,
        grid=(grid,),
        in_specs=[
            pl.BlockSpec((blk, D_IN), lambda i: (i, 0)),
            pl.BlockSpec((D_IN, 512), lambda i: (0, 0)),
            pl.BlockSpec((512, 16), lambda i: (0, 0)),
            pl.BlockSpec((512, 8), lambda i: (0, 0)),
            pl.BlockSpec((8, 16), lambda i: (0, 0)),
        ],
        out_specs=[
            pl.BlockSpec((8, blk, 64), lambda i: (0, i, 0)),
            pl.BlockSpec((blk, 16), lambda i: (i, 0)),
            pl.BlockSpec((8, blk, 64), lambda i: (0, i, 0)),
            pl.BlockSpec((blk, 16), lambda i: (i, 0)),
        ],
        out_shape=[
            jax.ShapeDtypeStruct((8, NPAD, 64), jnp.float32),
            jax.ShapeDtypeStruct((NPAD, 16), jnp.float32),
            jax.ShapeDtypeStruct((8, NPAD, 64), jnp.float32),
            jax.ShapeDtypeStruct((NPAD, 16), jnp.float32),
        ],
    )(xp, W1, M, MA, P)


_KB = 128
_TBE = (EPAD // (NC * NS)) // _KB


def _ring_edge_pass(src2d, dst2d, tbl_hbm, init_hbm, zeros16, acc_sh, tbl_sh,
                    ixs, ixd, gs, gd, gsem, dsem, ssem, isem, idsem,
                    compute, scatter_extra, extra_wait, c, s):
    wid = s * NC + c
    row0 = s * ROWS
    T = _TBE

    @pl.when(c == 0)
    def _():
        pltpu.sync_copy(init_hbm.at[pl.ds(row0, ROWS)],
                        acc_sh.at[pl.ds(row0, ROWS)])
    @pl.when(c != 0)
    def _():
        pltpu.sync_copy(zeros16.at[pl.ds(row0, ROWS)],
                        acc_sh.at[pl.ds(row0, ROWS)])
    pltpu.sync_copy(tbl_hbm.at[pl.ds(row0, ROWS)],
                    tbl_sh.at[pl.ds(row0, ROWS)])
    plsc.subcore_barrier()

    def idx_load(t, b, sync=False):
        if sync:
            pltpu.sync_copy(src2d.at[wid * T + t], ixs[b])
            pltpu.sync_copy(dst2d.at[wid * T + t], ixd[b])
        else:
            pltpu.async_copy(src2d.at[wid * T + t], ixs[b], isem.at[b])
            pltpu.async_copy(dst2d.at[wid * T + t], ixd[b], idsem.at[b])

    def idx_wait(t, b):
        pltpu.make_async_copy(src2d.at[wid * T + t], ixs[b],
                              isem.at[b]).wait()
        pltpu.make_async_copy(dst2d.at[wid * T + t], ixd[b],
                              idsem.at[b]).wait()

    def issue(t, b):
        pltpu.async_copy(tbl_sh.at[ixs[b]], gs[b], gsem.at[b])
        pltpu.async_copy(tbl_sh.at[ixd[b]], gd[b], dsem.at[b])

    for t0 in range(2):
        idx_load(t0, t0, sync=True)
        issue(t0, t0)
    idx_load(2, 2)
    idx_load(3, 3)

    def group(gg, _):
        for b in range(4):
            t = 4 * gg + b
            pltpu.make_async_copy(tbl_sh.at[ixs[b]], gs[b],
                                  gsem.at[b]).wait()
            pltpu.make_async_copy(tbl_sh.at[ixd[b]], gd[b],
                                  dsem.at[b]).wait()
            compute(b)
            pltpu.async_copy(gs[b], acc_sh.at[ixd[b]], ssem.at[b], add=True)
            scatter_extra(t, b)

            b2 = (b + 2) % 4
            @pl.when(t + 2 < T)
            def _():
                idx_wait(t + 2, b2)
                issue(t + 2, b2)

            b3 = (b + 3) % 4
            @pl.when((t >= 1) & (t + 3 < T))
            def _():
                pltpu.make_async_copy(gs[b3], acc_sh.at[ixd[b3]],
                                      ssem.at[b3]).wait()
                extra_wait(t - 1, b3)
                idx_load(t + 3, b3)
        return 0

    lax.fori_loop(0, T // 4, group, 0)
    for b in range(4):
        t = T - 4 + b
        pltpu.make_async_copy(gs[b], acc_sh.at[ixd[b]], ssem.at[b]).wait()
        extra_wait(t, b)
    plsc.subcore_barrier()


def _sc_b_body(src2d, dst2d, adu, selfden, zeros16,
               ex1_out, denp_out,
               den_sh, tbl_sh,
               is0, is1, is2, is3, id0, id1, id2, id3,
               g0, g1, g2, g3, d0, d1, d2, d3,
               gsem, dsem, ssem, xsem, isem, idsem):
    c = lax.axis_index("c")
    s = lax.axis_index("s")
    wid = s * NC + c
    gs = [g0, g1, g2, g3]
    gd = [d0, d1, d2, d3]
    lanes = _lane_iota()
    perm_idx = (lanes + 8) & 15

    def compute(b):
        @plsc.parallel_loop(0, _KB, step=1, unroll=2)
        def _edge(i):
            al = gs[b][i, :] + _take(gd[b][i, :], perm_idx)
            ex = jnp.exp(jnp.maximum(al, 0.2 * al))
            gs[b][i, :] = jnp.where(lanes < 8, ex, 0.0)

    def scatter_extra(t, b):
        base = (wid * _TBE + t) * _KB
        pltpu.async_copy(gs[b], ex1_out.at[pl.ds(base, _KB)], xsem.at[b])

    def extra_wait(t, b):
        base = (wid * _TBE + t) * _KB
        pltpu.make_async_copy(gs[b], ex1_out.at[pl.ds(base, _KB)],
                              xsem.at[b]).wait()

    _ring_edge_pass(src2d, dst2d, adu, selfden, zeros16, den_sh, tbl_sh,
                    [is0, is1, is2, is3], [id0, id1, id2, id3],
                    gs, gd, gsem, dsem, ssem, isem, idsem,
                    compute, scatter_extra, extra_wait, c, s)
    row0 = s * ROWS
    pltpu.sync_copy(den_sh.at[pl.ds(row0, ROWS)],
                    denp_out.at[pl.ds(c * NPAD + row0, ROWS)])


def _sc_b(srcp2d, dstp2d, adu, selfden, zeros16):
    f = pl.kernel(
        _sc_b_body,
        out_type=[
            jax.ShapeDtypeStruct((EPAD, 16), jnp.float32),
            jax.ShapeDtypeStruct((2 * NPAD, 16), jnp.float32),
        ],
        mesh=_mesh,
        scratch_types=(
            [pltpu.VMEM_SHARED((NPAD, 16), jnp.float32),
             pltpu.VMEM_SHARED((NPAD, 16), jnp.float32)]
            + [pltpu.VMEM((_KB,), jnp.int32) for _ in range(8)]
            + [pltpu.VMEM((_KB, 16), jnp.float32) for _ in range(8)]
            + [pltpu.SemaphoreType.DMA((4,)) for _ in range(6)]
        ),
        compiler_params=pltpu.CompilerParams(use_tc_tiling_on_sc=False),
    )
    return f(srcp2d, dstp2d, adu, selfden, zeros16)


KC = 128
_NCHUNK = (EPAD // NS) // KC
_NB = 4


def _sc_c_body(src2d, dst2d, h1flat, ex1, selfnumflat,
               num_out,
               acc_sh, tbl_sh,
               is0, is1, is2, is3, id0, id1, id2, id3,
               hr0, hr1, hr2, hr3, eb0, eb1, eb2, eb3,
               gsem, esem, ssem, isem, idsem):
    c = lax.axis_index("c")
    s = lax.axis_index("s")
    row0 = s * ROWS
    epersub = EPAD // NS
    hr = [hr0, hr1, hr2, hr3]
    eb = [eb0, eb1, eb2, eb3]
    ixs = [is0, is1, is2, is3]
    ixd = [id0, id1, id2, id3]
    T = _NCHUNK

    def idx_load(t, b, sync=False):
        if sync:
            pltpu.sync_copy(src2d.at[s * T + t], ixs[b])
            pltpu.sync_copy(dst2d.at[s * T + t], ixd[b])
        else:
            pltpu.async_copy(src2d.at[s * T + t], ixs[b], isem.at[b])
            pltpu.async_copy(dst2d.at[s * T + t], ixd[b], idsem.at[b])

    def idx_wait(t, b):
        pltpu.make_async_copy(src2d.at[s * T + t], ixs[b], isem.at[b]).wait()
        pltpu.make_async_copy(dst2d.at[s * T + t], ixd[b], idsem.at[b]).wait()

    for p in range(4):
        ch = p * NC + c
        off = ch * NPAD
        pltpu.sync_copy(h1flat.at[pl.ds(off + row0, ROWS)],
                        tbl_sh.at[pl.ds(row0, ROWS)])
        pltpu.sync_copy(selfnumflat.at[pl.ds(off + row0, ROWS)],
                        acc_sh.at[pl.ds(row0, ROWS)])
        plsc.subcore_barrier()
        hh = jnp.full((L,), ch, jnp.int32)

        def issue(t, b):
            pltpu.async_copy(tbl_sh.at[ixs[b]], hr[b], gsem.at[b])
            base = s * epersub + t * KC
            pltpu.async_copy(ex1.at[pl.ds(base, KC)], eb[b], esem.at[b])

        for t0 in range(2):
            idx_load(t0, t0, sync=True)
            issue(t0, t0)
        idx_load(2, 2)
        idx_load(3, 3)

        def group(gg, _):
            for b in range(_NB):
                t = _NB * gg + b
                base = s * epersub + t * KC
                pltpu.make_async_copy(tbl_sh.at[ixs[b]], hr[b],
                                      gsem.at[b]).wait()
                pltpu.make_async_copy(ex1.at[pl.ds(base, KC)], eb[b],
                                      esem.at[b]).wait()

                @plsc.parallel_loop(0, KC, step=1)
                def _edge(i):
                    sc = _take(eb[b][i, :], hh)
                    for j in range(4):
                        hr[b][i, pl.ds(j * L, L)] = \
                            hr[b][i, pl.ds(j * L, L)] * sc

                pltpu.async_copy(hr[b], acc_sh.at[ixd[b]],
                                 ssem.at[b], add=True)

                b2 = (b + 2) % _NB
                @pl.when(t + 2 < T)
                def _():
                    idx_wait(t + 2, b2)
                    issue(t + 2, b2)

                b3 = (b + 3) % _NB
                @pl.when((t >= 1) & (t + 3 < T))
                def _():
                    pltpu.make_async_copy(hr[b3], acc_sh.at[ixd[b3]],
                                          ssem.at[b3]).wait()
                    idx_load(t + 3, b3)
            return 0

        lax.fori_loop(0, T // _NB, group, 0)
        for b in range(_NB):
            pltpu.make_async_copy(hr[b], acc_sh.at[ixd[b]],
                                  ssem.at[b]).wait()
        plsc.subcore_barrier()
        pltpu.sync_copy(acc_sh.at[pl.ds(row0, ROWS)],
                        num_out.at[pl.ds(off + row0, ROWS)])
        plsc.subcore_barrier()


def _sc_c(src2d, dst2d, h1flat, ex1, selfnumflat):
    f = pl.kernel(
        _sc_c_body,
        out_type=jax.ShapeDtypeStruct((8 * NPAD, 64), jnp.float32),
        mesh=_mesh,
        scratch_types=(
            [pltpu.VMEM_SHARED((NPAD, 64), jnp.float32),
             pltpu.VMEM_SHARED((NPAD, 64), jnp.float32)]
            + [pltpu.VMEM((KC,), jnp.int32) for _ in range(8)]
            + [pltpu.VMEM((KC, 64), jnp.float32) for _ in range(4)]
            + [pltpu.VMEM((KC, 16), jnp.float32) for _ in range(4)]
            + [pltpu.SemaphoreType.DMA((_NB,)) for _ in range(5)]
        ),
        compiler_params=pltpu.CompilerParams(use_tc_tiling_on_sc=False),
    )
    return f(src2d, dst2d, h1flat, ex1, selfnumflat)


def _tc_f_body(n0_ref, n1_ref, n2_ref, n3_ref, n4_ref, n5_ref, n6_ref,
               n7_ref, dp0_ref, dp1_ref,
               r16_ref, b1_ref, w2p_ref, a_ref,
               rec2_ref, selfinit2_ref):
    den16 = dp0_ref[...] + dp1_ref[...]
    denb = jnp.dot(den16, r16_ref[...], preferred_element_type=jnp.float32)
    numn = jnp.concatenate(
        [n0_ref[...], n1_ref[...], n2_ref[...], n3_ref[...],
         n4_ref[...], n5_ref[...], n6_ref[...], n7_ref[...]], axis=1)
    o1 = numn / (denb + 1e-16) + b1_ref[...]
    h2pre = jnp.where(o1 > 0, o1, jnp.exp(jnp.minimum(o1, 0.0)) - 1.0)
    h2v = jnp.dot(h2pre, w2p_ref[...], preferred_element_type=jnp.float32)
    sc = jnp.dot(h2v, a_ref[...], preferred_element_type=jnp.float32)
    col = lax.broadcasted_iota(jnp.int32, h2v.shape, 1)
    rec2 = jnp.where(col == 7, 1.0, h2v + sc)
    rec2_ref[...] = rec2
    a2 = rec2[:, 8:9] + rec2[:, 9:10]
    exs2 = jnp.exp(jnp.where(a2 > 0, a2, 0.2 * a2))
    selfinit2_ref[...] = jnp.where(col < 8, rec2 * exs2, 0.0)


def _tc_f(num, denp, R16, b1r, W2p, A):
    blk = 256
    grid = NPAD // blk
    nspec = [pl.BlockSpec((blk, 64), (lambda i, c=c: (c * grid + i, 0)))
             for c in range(8)]
    dspec = [pl.BlockSpec((blk, 16), (lambda i, c=c: (c * grid + i, 0)))
             for c in range(2)]
    return pl.pallas_call(
        _tc_f_body,
        grid=(grid,),
        in_specs=nspec + dspec + [
            pl.BlockSpec((16, 512), lambda i: (0, 0)),
            pl.BlockSpec((1, 512), lambda i: (0, 0)),
            pl.BlockSpec((512, 16), lambda i: (0, 0)),
            pl.BlockSpec((16, 16), lambda i: (0, 0)),
        ],
        out_specs=[
            pl.BlockSpec((blk, 16), lambda i: (i, 0)),
            pl.BlockSpec((blk, 16), lambda i: (i, 0)),
        ],
        out_shape=[
            jax.ShapeDtypeStruct((NPAD, 16), jnp.float32),
            jax.ShapeDtypeStruct((NPAD, 16), jnp.float32),
        ],
    )(num, num, num, num, num, num, num, num, denp, denp, R16, b1r, W2p, A)


def _sc_e_body(src2d, dst2d, rec2, selfinit2, zeros16,
               accp_out,
               acc_sh, tbl_sh,
               is0, is1, is2, is3, id0, id1, id2, id3,
               g0, g1, g2, g3, d0, d1, d2, d3,
               gsem, dsem, ssem, isem, idsem):
    c = lax.axis_index("c")
    s = lax.axis_index("s")
    gs = [g0, g1, g2, g3]
    gd = [d0, d1, d2, d3]
    lanes = _lane_iota()
    i8 = jnp.full((L,), 8, jnp.int32)
    i9 = jnp.full((L,), 9, jnp.int32)

    def compute(b):
        @plsc.parallel_loop(0, _KB, step=1, unroll=2)
        def _edge(i):
            rs = gs[b][i, :]
            al = _take(rs, i8) + _take(gd[b][i, :], i9)
            ex = jnp.exp(jnp.maximum(al, 0.2 * al))
            gs[b][i, :] = jnp.where(lanes < 8, rs * ex, 0.0)

    def scatter_extra(t, b):
        pass

    def extra_wait(t, b):
        pass

    _ring_edge_pass(src2d, dst2d, rec2, selfinit2, zeros16, acc_sh, tbl_sh,
                    [is0, is1, is2, is3], [id0, id1, id2, id3],
                    gs, gd, gsem, dsem, ssem, isem, idsem,
                    compute, scatter_extra, extra_wait, c, s)
    row0 = s * ROWS
    pltpu.sync_copy(acc_sh.at[pl.ds(row0, ROWS)],
                    accp_out.at[pl.ds(c * NPAD + row0, ROWS)])


def _sc_e(srcp2d, dstp2d, rec2, selfinit2, zeros16):
    f = pl.kernel(
        _sc_e_body,
        out_type=jax.ShapeDtypeStruct((2 * NPAD, 16), jnp.float32),
        mesh=_mesh,
        scratch_types=(
            [pltpu.VMEM_SHARED((NPAD, 16), jnp.float32),
             pltpu.VMEM_SHARED((NPAD, 16), jnp.float32)]
            + [pltpu.VMEM((_KB,), jnp.int32) for _ in range(8)]
            + [pltpu.VMEM((_KB, 16), jnp.float32) for _ in range(8)]
            + [pltpu.SemaphoreType.DMA((4,)) for _ in range(5)]
        ),
        compiler_params=pltpu.CompilerParams(use_tc_tiling_on_sc=False),
    )
    return f(srcp2d, dstp2d, rec2, selfinit2, zeros16)


def _tc_g_body(a0_ref, a1_ref, b2_ref, out_ref):
    acc = a0_ref[...] + a1_ref[...]
    o2 = acc / (acc[:, 7:8] + 1e-16) + b2_ref[...]
    col = lax.broadcasted_iota(jnp.int32, o2.shape, 1)
    o2m = jnp.where(col < 7, o2, -jnp.inf)
    m = jnp.max(o2m, axis=1, keepdims=True)
    ex = jnp.where(col < 7, jnp.exp(o2m - m), 0.0)
    ssum = jnp.sum(ex, axis=1, keepdims=True)
    out_ref[...] = o2m - m - jnp.log(ssum)


def _tc_g(accp, b2p):
    blk = 256
    grid = NPAD // blk
    aspec = [pl.BlockSpec((blk, 16), (lambda i, c=c: (c * grid + i, 0)))
             for c in range(2)]
    return pl.pallas_call(
        _tc_g_body,
        grid=(grid,),
        in_specs=aspec + [pl.BlockSpec((1, 16), lambda i: (0, 0))],
        out_specs=pl.BlockSpec((blk, 16), lambda i: (i, 0)),
        out_shape=jax.ShapeDtypeStruct((NPAD, 16), jnp.float32),
    )(accp, accp, b2p)


@jax.jit
def _run(x, src, dst, W1, M, MA, P, R16, b1r, W2p, A, b2p, zeros16):
    xp = jnp.pad(x, ((0, NPAD - N), (0, 0)))
    srcp = jnp.concatenate([src, jnp.full((EPAD - E,), N, jnp.int32)])
    dstp = jnp.concatenate([dst, jnp.full((EPAD - E,), N, jnp.int32)])

    h1t, adu, selfnum, selfden = _tc_a(xp, W1, M, MA, P)
    h1flat = h1t.reshape(8 * NPAD, 64)
    selfnumflat = selfnum.reshape(8 * NPAD, 64)

    sb = srcp.reshape(EPAD // _KB, _KB)
    db = dstp.reshape(EPAD // _KB, _KB)
    ex1, denp = _sc_b(sb, db, adu, selfden, zeros16)
    num = _sc_c(srcp.reshape(EPAD // KC, KC), dstp.reshape(EPAD // KC, KC),
                h1flat, ex1, selfnumflat)
    rec2, selfinit2 = _tc_f(num, denp, R16, b1r, W2p, A)
    accp = _sc_e(sb, db, rec2, selfinit2, zeros16)
    out = _tc_g(accp, b2p)
    return out[:N, :7]


def kernel(x, edge_index, W1, att_src1, att_dst1, b1, W2, att_src2,
           att_dst2, b2):
    as1 = att_src1[0].astype(jnp.float32)
    ad1 = att_dst1[0].astype(jnp.float32)
    a2s = att_src2[0, 0].astype(jnp.float32)
    a2d = att_dst2[0, 0].astype(jnp.float32)

    hsel = np.repeat(np.arange(8), 64)
    eye8 = (hsel[:, None] == np.arange(8)[None, :]).astype(np.float32)
    as_flat = as1.reshape(-1)
    ad_flat = ad1.reshape(-1)
    M = jnp.concatenate([eye8 * as_flat[:, None],
                         eye8 * ad_flat[:, None]], axis=1)
    MA = eye8 * (as_flat + ad_flat)[:, None]
    R16 = jnp.concatenate([jnp.asarray(eye8.T),
                           jnp.zeros((8, 512), jnp.float32)], axis=0)
    P = jnp.concatenate([jnp.eye(8, dtype=jnp.float32),
                         jnp.zeros((8, 8), jnp.float32)], axis=1)
    W2p = jnp.pad(W2.astype(jnp.float32), ((0, 0), (0, 16 - C2)))
    A = jnp.zeros((16, 16), jnp.float32)
    A = A.at[0:7, 8].set(a2s).at[0:7, 9].set(a2d)
    b1r = b1.astype(jnp.float32).reshape(1, 512)
    b2p = jnp.pad(b2.astype(jnp.float32), (0, 9)).reshape(1, 16)
    zeros16 = jnp.zeros((NPAD, 16), jnp.float32)

    src = edge_index[0].astype(jnp.int32)
    dst = edge_index[1].astype(jnp.int32)
    return _run(x, src, dst, W1, M, MA, P, R16,
                b1r, W2p, A, b2p, zeros16)


# speedup vs baseline: 1.6087x; 1.1427x over previous
import functools
import jax
import jax.numpy as jnp
import numpy as np
from jax import lax
from jax.experimental import pallas as pl
from jax.experimental.pallas import tpu as pltpu
from jax.experimental.pallas import tpu_sc as plsc

N = 10000
E = 160000
D_IN = 256
H1, C1 = 8, 64
H2, C2 = 1, 7

NPAD = 10240
EPAD = 163840
NC, NS, L = 2, 16, 16
ROWS = NPAD // NS
K = 128

_mesh = plsc.VectorSubcoreMesh(core_axis_name="c", subcore_axis_name="s")


def _take(v, idx):
    return lax.gather(
        v, idx[:, None],
        dimension_numbers=lax.GatherDimensionNumbers(
            offset_dims=(), collapsed_slice_dims=(0,), start_index_map=(0,)),
        slice_sizes=(1,),
        mode=lax.GatherScatterMode.PROMISE_IN_BOUNDS)


def _lane_iota():
    return lax.iota(jnp.int32, L)


def _tc_a_body(x_ref, w1_ref, m_ref, ma_ref, r_ref, p_ref,
               h1_ref, adu_ref, selfnum_ref, selfden_ref):
    h = jnp.dot(x_ref[...], w1_ref[...], preferred_element_type=jnp.float32)
    adu_ref[...] = jnp.dot(h, m_ref[...], preferred_element_type=jnp.float32)
    alpha = jnp.dot(h, ma_ref[...], preferred_element_type=jnp.float32)
    exs = jnp.exp(jnp.where(alpha > 0, alpha, 0.2 * alpha))
    selfden_ref[...] = jnp.dot(exs, p_ref[...],
                               preferred_element_type=jnp.float32)
    exrep = jnp.dot(exs, r_ref[...], preferred_element_type=jnp.float32)
    for j in range(4):
        h1_ref[j] = h[:, 128 * j:128 * (j + 1)]
        selfnum_ref[j] = h1_ref[j] * exrep[:, 128 * j:128 * (j + 1)]


def _tc_a(xp, W1, M, MA, R, P):
    blk = 256
    grid = NPAD // blk
    return pl.pallas_call(
        _tc_a_body,
        grid=(grid,),
        in_specs=[
            pl.BlockSpec((blk, D_IN), lambda i: (i, 0)),
            pl.BlockSpec((D_IN, 512), lambda i: (0, 0)),
            pl.BlockSpec((512, 16), lambda i: (0, 0)),
            pl.BlockSpec((512, 8), lambda i: (0, 0)),
            pl.BlockSpec((8, 512), lambda i: (0, 0)),
            pl.BlockSpec((8, 16), lambda i: (0, 0)),
        ],
        out_specs=[
            pl.BlockSpec((4, blk, 128), lambda i: (0, i, 0)),
            pl.BlockSpec((blk, 16), lambda i: (i, 0)),
            pl.BlockSpec((4, blk, 128), lambda i: (0, i, 0)),
            pl.BlockSpec((blk, 16), lambda i: (i, 0)),
        ],
        out_shape=[
            jax.ShapeDtypeStruct((4, NPAD, 128), jnp.float32),
            jax.ShapeDtypeStruct((NPAD, 16), jnp.float32),
            jax.ShapeDtypeStruct((4, NPAD, 128), jnp.float32),
            jax.ShapeDtypeStruct((NPAD, 16), jnp.float32),
        ],
    )(xp, W1, M, MA, R, P)


_KB = 128
_TBE = (EPAD // (NC * NS)) // _KB


def _ring_edge_pass(src2d, dst2d, tbl_hbm, init_hbm, zeros16, acc_sh, tbl_sh,
                    ixs, ixd, gs, gd, gsem, dsem, ssem, isem, idsem,
                    compute, scatter_extra, extra_wait, c, s):
    wid = s * NC + c
    row0 = s * ROWS
    T = _TBE

    @pl.when(c == 0)
    def _():
        pltpu.sync_copy(init_hbm.at[pl.ds(row0, ROWS)],
                        acc_sh.at[pl.ds(row0, ROWS)])
    @pl.when(c != 0)
    def _():
        pltpu.sync_copy(zeros16.at[pl.ds(row0, ROWS)],
                        acc_sh.at[pl.ds(row0, ROWS)])
    pltpu.sync_copy(tbl_hbm.at[pl.ds(row0, ROWS)],
                    tbl_sh.at[pl.ds(row0, ROWS)])
    plsc.subcore_barrier()

    def idx_load(t, b, sync=False):
        if sync:
            pltpu.sync_copy(src2d.at[wid * T + t], ixs[b])
            pltpu.sync_copy(dst2d.at[wid * T + t], ixd[b])
        else:
            pltpu.async_copy(src2d.at[wid * T + t], ixs[b], isem.at[b])
            pltpu.async_copy(dst2d.at[wid * T + t], ixd[b], idsem.at[b])

    def idx_wait(t, b):
        pltpu.make_async_copy(src2d.at[wid * T + t], ixs[b],
                              isem.at[b]).wait()
        pltpu.make_async_copy(dst2d.at[wid * T + t], ixd[b],
                              idsem.at[b]).wait()

    def issue(t, b):
        pltpu.async_copy(tbl_sh.at[ixs[b]], gs[b], gsem.at[b])
        pltpu.async_copy(tbl_sh.at[ixd[b]], gd[b], dsem.at[b])

    for t0 in range(2):
        idx_load(t0, t0, sync=True)
        issue(t0, t0)
    idx_load(2, 2)
    idx_load(3, 3)

    def group(gg, _):
        for b in range(4):
            t = 4 * gg + b
            pltpu.make_async_copy(tbl_sh.at[ixs[b]], gs[b],
                                  gsem.at[b]).wait()
            pltpu.make_async_copy(tbl_sh.at[ixd[b]], gd[b],
                                  dsem.at[b]).wait()
            compute(b)
            pltpu.async_copy(gs[b], acc_sh.at[ixd[b]], ssem.at[b], add=True)
            scatter_extra(t, b)

            b2 = (b + 2) % 4
            @pl.when(t + 2 < T)
            def _():
                idx_wait(t + 2, b2)
                issue(t + 2, b2)

            b3 = (b + 3) % 4
            @pl.when((t >= 1) & (t + 3 < T))
            def _():
                pltpu.make_async_copy(gs[b3], acc_sh.at[ixd[b3]],
                                      ssem.at[b3]).wait()
                extra_wait(t - 1, b3)
                idx_load(t + 3, b3)
        return 0

    lax.fori_loop(0, T // 4, group, 0)
    for b in range(4):
        t = T - 4 + b
        pltpu.make_async_copy(gs[b], acc_sh.at[ixd[b]], ssem.at[b]).wait()
        extra_wait(t, b)
    plsc.subcore_barrier()


def _sc_b_body(src2d, dst2d, adu, selfden, zeros16,
               ex1_out, denp_out,
               den_sh, tbl_sh,
               is0, is1, is2, is3, id0, id1, id2, id3,
               g0, g1, g2, g3, d0, d1, d2, d3,
               gsem, dsem, ssem, xsem, isem, idsem):
    c = lax.axis_index("c")
    s = lax.axis_index("s")
    wid = s * NC + c
    gs = [g0, g1, g2, g3]
    gd = [d0, d1, d2, d3]
    lanes = _lane_iota()
    perm_idx = (lanes + 8) & 15

    def compute(b):
        @plsc.parallel_loop(0, _KB, step=1, unroll=2)
        def _edge(i):
            al = gs[b][i, :] + _take(gd[b][i, :], perm_idx)
            ex = jnp.exp(jnp.maximum(al, 0.2 * al))
            gs[b][i, :] = jnp.where(lanes < 8, ex, 0.0)

    def scatter_extra(t, b):
        base = (wid * _TBE + t) * _KB
        pltpu.async_copy(gs[b], ex1_out.at[pl.ds(base, _KB)], xsem.at[b])

    def extra_wait(t, b):
        base = (wid * _TBE + t) * _KB
        pltpu.make_async_copy(gs[b], ex1_out.at[pl.ds(base, _KB)],
                              xsem.at[b]).wait()

    _ring_edge_pass(src2d, dst2d, adu, selfden, zeros16, den_sh, tbl_sh,
                    [is0, is1, is2, is3], [id0, id1, id2, id3],
                    gs, gd, gsem, dsem, ssem, isem, idsem,
                    compute, scatter_extra, extra_wait, c, s)
    row0 = s * ROWS
    pltpu.sync_copy(den_sh.at[pl.ds(row0, ROWS)],
                    denp_out.at[pl.ds(c * NPAD + row0, ROWS)])


def _sc_b(srcp2d, dstp2d, adu, selfden, zeros16):
    f = pl.kernel(
        _sc_b_body,
        out_type=[
            jax.ShapeDtypeStruct((EPAD, 16), jnp.float32),
            jax.ShapeDtypeStruct((2 * NPAD, 16), jnp.float32),
        ],
        mesh=_mesh,
        scratch_types=(
            [pltpu.VMEM_SHARED((NPAD, 16), jnp.float32),
             pltpu.VMEM_SHARED((NPAD, 16), jnp.float32)]
            + [pltpu.VMEM((_KB,), jnp.int32) for _ in range(8)]
            + [pltpu.VMEM((_KB, 16), jnp.float32) for _ in range(8)]
            + [pltpu.SemaphoreType.DMA((4,)) for _ in range(6)]
        ),
        compiler_params=pltpu.CompilerParams(use_tc_tiling_on_sc=False),
    )
    return f(srcp2d, dstp2d, adu, selfden, zeros16)


KC = 128
_NCHUNK = (EPAD // NS) // KC
_NB = 4


def _sc_c_body(src2d, dst2d, h1flat, ex1, selfnumflat,
               num_out,
               acc_sh, tbl_sh,
               is0, is1, is2, is3, id0, id1, id2, id3,
               hr0, hr1, hr2, hr3, eb0, eb1, eb2, eb3,
               gsem, esem, ssem, isem, idsem):
    c = lax.axis_index("c")
    s = lax.axis_index("s")
    row0 = s * ROWS
    epersub = EPAD // NS
    hr = [hr0, hr1, hr2, hr3]
    eb = [eb0, eb1, eb2, eb3]
    ixs = [is0, is1, is2, is3]
    ixd = [id0, id1, id2, id3]
    T = _NCHUNK

    def idx_load(t, b, sync=False):
        if sync:
            pltpu.sync_copy(src2d.at[s * T + t], ixs[b])
            pltpu.sync_copy(dst2d.at[s * T + t], ixd[b])
        else:
            pltpu.async_copy(src2d.at[s * T + t], ixs[b], isem.at[b])
            pltpu.async_copy(dst2d.at[s * T + t], ixd[b], idsem.at[b])

    def idx_wait(t, b):
        pltpu.make_async_copy(src2d.at[s * T + t], ixs[b], isem.at[b]).wait()
        pltpu.make_async_copy(dst2d.at[s * T + t], ixd[b], idsem.at[b]).wait()

    for p in range(4):
        ch = p * NC + c
        colh = 64 * c
        pltpu.sync_copy(h1flat.at[pl.ds(p * NPAD + row0, ROWS),
                                  pl.ds(colh, 64)],
                        tbl_sh.at[pl.ds(row0, ROWS)])
        pltpu.sync_copy(selfnumflat.at[pl.ds(p * NPAD + row0, ROWS),
                                       pl.ds(colh, 64)],
                        acc_sh.at[pl.ds(row0, ROWS)])
        plsc.subcore_barrier()
        hh = jnp.full((L,), ch, jnp.int32)

        def issue(t, b):
            pltpu.async_copy(tbl_sh.at[ixs[b]], hr[b], gsem.at[b])
            base = s * epersub + t * KC
            pltpu.async_copy(ex1.at[pl.ds(base, KC)], eb[b], esem.at[b])

        for t0 in range(2):
            idx_load(t0, t0, sync=True)
            issue(t0, t0)
        idx_load(2, 2)
        idx_load(3, 3)

        def group(gg, _):
            for b in range(_NB):
                t = _NB * gg + b
                base = s * epersub + t * KC
                pltpu.make_async_copy(tbl_sh.at[ixs[b]], hr[b],
                                      gsem.at[b]).wait()
                pltpu.make_async_copy(ex1.at[pl.ds(base, KC)], eb[b],
                                      esem.at[b]).wait()

                @plsc.parallel_loop(0, KC, step=1)
                def _edge(i):
                    sc = _take(eb[b][i, :], hh)
                    for j in range(4):
                        hr[b][i, pl.ds(j * L, L)] = \
                            hr[b][i, pl.ds(j * L, L)] * sc

                pltpu.async_copy(hr[b], acc_sh.at[ixd[b]],
                                 ssem.at[b], add=True)

                b2 = (b + 2) % _NB
                @pl.when(t + 2 < T)
                def _():
                    idx_wait(t + 2, b2)
                    issue(t + 2, b2)

                b3 = (b + 3) % _NB
                @pl.when((t >= 1) & (t + 3 < T))
                def _():
                    pltpu.make_async_copy(hr[b3], acc_sh.at[ixd[b3]],
                                          ssem.at[b3]).wait()
                    idx_load(t + 3, b3)
            return 0

        lax.fori_loop(0, T // _NB, group, 0)
        for b in range(_NB):
            pltpu.make_async_copy(hr[b], acc_sh.at[ixd[b]],
                                  ssem.at[b]).wait()
        plsc.subcore_barrier()
        pltpu.sync_copy(acc_sh.at[pl.ds(row0, ROWS)],
                        num_out.at[pl.ds(p * NPAD + row0, ROWS),
                                   pl.ds(colh, 64)])
        plsc.subcore_barrier()


def _sc_c(src2d, dst2d, h1flat, ex1, selfnumflat):
    f = pl.kernel(
        _sc_c_body,
        out_type=jax.ShapeDtypeStruct((4 * NPAD, 128), jnp.float32),
        mesh=_mesh,
        scratch_types=(
            [pltpu.VMEM_SHARED((NPAD, 64), jnp.float32),
             pltpu.VMEM_SHARED((NPAD, 64), jnp.float32)]
            + [pltpu.VMEM((KC,), jnp.int32) for _ in range(8)]
            + [pltpu.VMEM((KC, 64), jnp.float32) for _ in range(4)]
            + [pltpu.VMEM((KC, 16), jnp.float32) for _ in range(4)]
            + [pltpu.SemaphoreType.DMA((_NB,)) for _ in range(5)]
        ),
        compiler_params=pltpu.CompilerParams(use_tc_tiling_on_sc=False),
    )
    return f(src2d, dst2d, h1flat, ex1, selfnumflat)


def _tc_f_body(n0_ref, n1_ref, n2_ref, n3_ref, dp0_ref, dp1_ref,
               r16_ref, b1_ref, w2p_ref, a_ref,
               rec2_ref, selfinit2_ref):
    den16 = dp0_ref[...] + dp1_ref[...]
    denb = jnp.dot(den16, r16_ref[...], preferred_element_type=jnp.float32)
    numn = jnp.concatenate(
        [n0_ref[...], n1_ref[...], n2_ref[...], n3_ref[...]], axis=1)
    o1 = numn / (denb + 1e-16) + b1_ref[...]
    h2pre = jnp.where(o1 > 0, o1, jnp.exp(jnp.minimum(o1, 0.0)) - 1.0)
    h2v = jnp.dot(h2pre, w2p_ref[...], preferred_element_type=jnp.float32)
    sc = jnp.dot(h2v, a_ref[...], preferred_element_type=jnp.float32)
    col = lax.broadcasted_iota(jnp.int32, h2v.shape, 1)
    rec2 = jnp.where(col == 7, 1.0, h2v + sc)
    rec2_ref[...] = rec2
    a2 = rec2[:, 8:9] + rec2[:, 9:10]
    exs2 = jnp.exp(jnp.where(a2 > 0, a2, 0.2 * a2))
    selfinit2_ref[...] = jnp.where(col < 8, rec2 * exs2, 0.0)


def _tc_f(num, denp, R16, b1r, W2p, A):
    blk = 256
    grid = NPAD // blk
    nspec = [pl.BlockSpec((blk, 128), (lambda i, c=c: (c * grid + i, 0)))
             for c in range(4)]
    dspec = [pl.BlockSpec((blk, 16), (lambda i, c=c: (c * grid + i, 0)))
             for c in range(2)]
    return pl.pallas_call(
        _tc_f_body,
        grid=(grid,),
        in_specs=nspec + dspec + [
            pl.BlockSpec((16, 512), lambda i: (0, 0)),
            pl.BlockSpec((1, 512), lambda i: (0, 0)),
            pl.BlockSpec((512, 16), lambda i: (0, 0)),
            pl.BlockSpec((16, 16), lambda i: (0, 0)),
        ],
        out_specs=[
            pl.BlockSpec((blk, 16), lambda i: (i, 0)),
            pl.BlockSpec((blk, 16), lambda i: (i, 0)),
        ],
        out_shape=[
            jax.ShapeDtypeStruct((NPAD, 16), jnp.float32),
            jax.ShapeDtypeStruct((NPAD, 16), jnp.float32),
        ],
    )(num, num, num, num, denp, denp, R16, b1r, W2p, A)


def _sc_e_body(src2d, dst2d, rec2, selfinit2, zeros16,
               accp_out,
               acc_sh, tbl_sh,
               is0, is1, is2, is3, id0, id1, id2, id3,
               g0, g1, g2, g3, d0, d1, d2, d3,
               gsem, dsem, ssem, isem, idsem):
    c = lax.axis_index("c")
    s = lax.axis_index("s")
    gs = [g0, g1, g2, g3]
    gd = [d0, d1, d2, d3]
    lanes = _lane_iota()
    i8 = jnp.full((L,), 8, jnp.int32)
    i9 = jnp.full((L,), 9, jnp.int32)

    def compute(b):
        @plsc.parallel_loop(0, _KB, step=1, unroll=2)
        def _edge(i):
            rs = gs[b][i, :]
            al = _take(rs, i8) + _take(gd[b][i, :], i9)
            ex = jnp.exp(jnp.maximum(al, 0.2 * al))
            gs[b][i, :] = jnp.where(lanes < 8, rs * ex, 0.0)

    def scatter_extra(t, b):
        pass

    def extra_wait(t, b):
        pass

    _ring_edge_pass(src2d, dst2d, rec2, selfinit2, zeros16, acc_sh, tbl_sh,
                    [is0, is1, is2, is3], [id0, id1, id2, id3],
                    gs, gd, gsem, dsem, ssem, isem, idsem,
                    compute, scatter_extra, extra_wait, c, s)
    row0 = s * ROWS
    pltpu.sync_copy(acc_sh.at[pl.ds(row0, ROWS)],
                    accp_out.at[pl.ds(c * NPAD + row0, ROWS)])


def _sc_e(srcp2d, dstp2d, rec2, selfinit2, zeros16):
    f = pl.kernel(
        _sc_e_body,
        out_type=jax.ShapeDtypeStruct((2 * NPAD, 16), jnp.float32),
        mesh=_mesh,
        scratch_types=(
            [pltpu.VMEM_SHARED((NPAD, 16), jnp.float32),
             pltpu.VMEM_SHARED((NPAD, 16), jnp.float32)]
            + [pltpu.VMEM((_KB,), jnp.int32) for _ in range(8)]
            + [pltpu.VMEM((_KB, 16), jnp.float32) for _ in range(8)]
            + [pltpu.SemaphoreType.DMA((4,)) for _ in range(5)]
        ),
        compiler_params=pltpu.CompilerParams(use_tc_tiling_on_sc=False),
    )
    return f(srcp2d, dstp2d, rec2, selfinit2, zeros16)


def _tc_g_body(a0_ref, a1_ref, b2_ref, out_ref):
    acc = a0_ref[...] + a1_ref[...]
    o2 = acc / (acc[:, 7:8] + 1e-16) + b2_ref[...]
    col = lax.broadcasted_iota(jnp.int32, o2.shape, 1)
    o2m = jnp.where(col < 7, o2, -jnp.inf)
    m = jnp.max(o2m, axis=1, keepdims=True)
    ex = jnp.where(col < 7, jnp.exp(o2m - m), 0.0)
    ssum = jnp.sum(ex, axis=1, keepdims=True)
    out_ref[...] = o2m - m - jnp.log(ssum)


def _tc_g(accp, b2p):
    blk = 256
    grid = NPAD // blk
    aspec = [pl.BlockSpec((blk, 16), (lambda i, c=c: (c * grid + i, 0)))
             for c in range(2)]
    return pl.pallas_call(
        _tc_g_body,
        grid=(grid,),
        in_specs=aspec + [pl.BlockSpec((1, 16), lambda i: (0, 0))],
        out_specs=pl.BlockSpec((blk, 16), lambda i: (i, 0)),
        out_shape=jax.ShapeDtypeStruct((NPAD, 16), jnp.float32),
    )(accp, accp, b2p)


@jax.jit
def _run(x, src, dst, W1, M, MA, R, P, R16, b1r, W2p, A, b2p, zeros16):
    xp = jnp.pad(x, ((0, NPAD - N), (0, 0)))
    srcp = jnp.concatenate([src, jnp.full((EPAD - E,), N, jnp.int32)])
    dstp = jnp.concatenate([dst, jnp.full((EPAD - E,), N, jnp.int32)])

    h1t, adu, selfnum, selfden = _tc_a(xp, W1, M, MA, R, P)
    h1flat = h1t.reshape(4 * NPAD, 128)
    selfnumflat = selfnum.reshape(4 * NPAD, 128)

    sb = srcp.reshape(EPAD // _KB, _KB)
    db = dstp.reshape(EPAD // _KB, _KB)
    ex1, denp = _sc_b(sb, db, adu, selfden, zeros16)
    num = _sc_c(srcp.reshape(EPAD // KC, KC), dstp.reshape(EPAD // KC, KC),
                h1flat, ex1, selfnumflat)
    rec2, selfinit2 = _tc_f(num, denp, R16, b1r, W2p, A)
    accp = _sc_e(sb, db, rec2, selfinit2, zeros16)
    out = _tc_g(accp, b2p)
    return out[:N, :7]


def kernel(x, edge_index, W1, att_src1, att_dst1, b1, W2, att_src2,
           att_dst2, b2):
    as1 = att_src1[0].astype(jnp.float32)
    ad1 = att_dst1[0].astype(jnp.float32)
    a2s = att_src2[0, 0].astype(jnp.float32)
    a2d = att_dst2[0, 0].astype(jnp.float32)

    hsel = np.repeat(np.arange(8), 64)
    eye8 = (hsel[:, None] == np.arange(8)[None, :]).astype(np.float32)
    as_flat = as1.reshape(-1)
    ad_flat = ad1.reshape(-1)
    M = jnp.concatenate([eye8 * as_flat[:, None],
                         eye8 * ad_flat[:, None]], axis=1)
    MA = eye8 * (as_flat + ad_flat)[:, None]
    R = jnp.asarray(eye8.T)
    R16 = jnp.concatenate([R, jnp.zeros((8, 512), jnp.float32)], axis=0)
    P = jnp.concatenate([jnp.eye(8, dtype=jnp.float32),
                         jnp.zeros((8, 8), jnp.float32)], axis=1)
    W2p = jnp.pad(W2.astype(jnp.float32), ((0, 0), (0, 16 - C2)))
    A = jnp.zeros((16, 16), jnp.float32)
    A = A.at[0:7, 8].set(a2s).at[0:7, 9].set(a2d)
    b1r = b1.astype(jnp.float32).reshape(1, 512)
    b2p = jnp.pad(b2.astype(jnp.float32), (0, 9)).reshape(1, 16)
    zeros16 = jnp.zeros((NPAD, 16), jnp.float32)

    src = edge_index[0].astype(jnp.int32)
    dst = edge_index[1].astype(jnp.int32)
    return _run(x, src, dst, W1, M, MA, R, P, R16,
                b1r, W2p, A, b2p, zeros16)
